# baseline (speedup 1.0000x reference)
import functools

import jax
import jax.numpy as jnp
from jax import lax
from jax.experimental import pallas as pl
from jax.experimental.pallas import tpu as pltpu
from jax.experimental.pallas import tpu_sc as plsc

_B, _N, _CIN = 4, 4096, 67
_S, _K, _F = 16, 15, 64
_KP_EXTEND = 0.1
_ALPHA = 0.3
_R2MAX = (2.0 * _KP_EXTEND) ** 2
_BIG = 1e30
_CP = 128
_NW = 32
_RPW = (_B * _N) // _NW
_WPB = _N // _RPW
_GQ = 8
_NG = _RPW // _GQ
_NCHUNK = _N // 16


def _sc_select_gather(x_hbm, y_hbm, z_hbm, tbl_hbm,
                      grouped_hbm, relx_hbm, rely_hbm, relz_hbm, seld2_hbm,
                      xv, yv, zv, sd2, sidx, idxsel, selk,
                      rxb, ryb, rzb, rows, sem):
    wid = lax.axis_index("s") * 2 + lax.axis_index("c")
    b = wid >> 3
    nbase = (wid & (_WPB - 1)) * _RPW
    gbase = b * _N + nbase

    pltpu.sync_copy(x_hbm.at[pl.ds(b * _N, _N)], xv)
    pltpu.sync_copy(y_hbm.at[pl.ds(b * _N, _N)], yv)
    pltpu.sync_copy(z_hbm.at[pl.ds(b * _N, _N)], zv)

    iota = lax.iota(jnp.int32, 16)

    def row_body(i, carry):
        n = nbase + i
        nsplat = jnp.full((16,), n, jnp.int32)
        qx = plsc.load_gather(xv, [nsplat])
        qy = plsc.load_gather(yv, [nsplat])
        qz = plsc.load_gather(zv, [nsplat])

        def scan_body(c, cnt):
            c16 = c * 16
            xc = xv[pl.ds(c16, 16)]
            yc = yv[pl.ds(c16, 16)]
            zc = zv[pl.ds(c16, 16)]
            dx = xc - qx
            dy = yc - qy
            dz = zc - qz
            d2 = (dx * dx + dy * dy) + dz * dz
            m = d2 < _R2MAX
            pos = plsc.cumsum(m.astype(jnp.int32)) - 1 + cnt
            plsc.store_scatter(sd2, [pos], d2, mask=m)
            plsc.store_scatter(sidx, [pos], iota + c16, mask=m)
            pc = plsc.all_reduce_population_count(m)
            if pc.ndim == 0:
                pc = jnp.full((16,), pc, jnp.int32)
            return cnt + pc

        cnt = lax.fori_loop(0, _NCHUNK, scan_body, jnp.zeros((16,), jnp.int32))
        cnt_s = jnp.max(cnt)

        padpos = cnt + iota
        plsc.store_scatter(sd2, [padpos], jnp.full((16,), _BIG, jnp.float32))
        plsc.store_scatter(sidx, [padpos], nsplat)

        runk, runv = plsc.sort_key_val(sd2[pl.ds(0, 16)], sidx[pl.ds(0, 16)])

        def merge_body(j, kv):
            rk, rv = kv
            bs = plsc.sort_key_val(sd2[pl.ds(j * 16, 16)], sidx[pl.ds(j * 16, 16)])
            rbk = lax.rev(bs[0], (0,))
            rbv = lax.rev(bs[1], (0,))
            keep = rk <= rbk
            mk = jnp.where(keep, rk, rbk)
            mv = jnp.where(keep, rv, rbv)
            sk = plsc.sort_key_val(mk, mv)
            return (sk[0], sk[1])

        nch = (cnt_s + 15) >> 4
        runk, runv = lax.fori_loop(1, nch, merge_body, (runk, runv))

        relx = plsc.load_gather(xv, [runv]) - qx
        rely = plsc.load_gather(yv, [runv]) - qy
        relz = plsc.load_gather(zv, [runv]) - qz

        o = i * 16
        rxb[pl.ds(o, 16)] = relx
        ryb[pl.ds(o, 16)] = rely
        rzb[pl.ds(o, 16)] = relz
        selk[pl.ds(o, 16)] = runk
        idxsel[pl.ds(o, 16)] = runv + b * _N
        return carry

    lax.fori_loop(0, _RPW, row_body, 0)

    ob = gbase * 16
    pltpu.sync_copy(rxb, relx_hbm.at[pl.ds(ob, _RPW * 16)])
    pltpu.sync_copy(ryb, rely_hbm.at[pl.ds(ob, _RPW * 16)])
    pltpu.sync_copy(rzb, relz_hbm.at[pl.ds(ob, _RPW * 16)])
    pltpu.sync_copy(selk, seld2_hbm.at[pl.ds(ob, _RPW * 16)])

    def gather_body(g, carry):
        gi = g * _GQ * 16
        pltpu.async_copy(tbl_hbm.at[idxsel.at[pl.ds(gi, _GQ * 16)]], rows, sem).wait()
        pltpu.sync_copy(rows, grouped_hbm.at[pl.ds(gbase * 16 + gi, _GQ * 16)])
        return carry

    lax.fori_loop(0, _NG, gather_body, 0)


def _tc_weight_reduce(grouped_ref, relx_ref, rely_ref, relz_ref, seld2_ref,
                      kp_ref, w_ref, out_ref):
    g = grouped_ref[...]
    relx = relx_ref[...]
    rely = rely_ref[...]
    relz = relz_ref[...]
    validf = jnp.where(seld2_ref[...] < _R2MAX, 1.0, 0.0)

    nq = relx.shape[0]
    gs = [g[:, s, :_CIN] for s in range(_S)]
    out = jnp.zeros((nq, _F), jnp.float32)
    for k in range(_K):
        ax, ay, az = kp_ref[k, 0], kp_ref[k, 1], kp_ref[k, 2]
        dxk = relx - ax
        dyk = rely - ay
        dzk = relz - az
        dk2 = (dxk * dxk + dyk * dyk) + dzk * dzk
        dist = jnp.sqrt(dk2 + 1e-12)
        infl = jnp.maximum(0.0, 1.0 - dist * (1.0 / _KP_EXTEND)) * validf
        hk = gs[0] * infl[:, 0][:, None]
        for s in range(1, _S):
            hk = hk + gs[s] * infl[:, s][:, None]
        out = out + jax.lax.dot_general(hk, w_ref[k], (((1,), (0,)), ((), ())),
                                        preferred_element_type=jnp.float32)
    out_ref[...] = jnp.where(out > 0, out, _ALPHA * out)


_NQT = 128


def kernel(inputs, kernel_points, W):
    xyz = inputs[..., :3]
    xf = xyz[..., 0].reshape(-1)
    yf = xyz[..., 1].reshape(-1)
    zf = xyz[..., 2].reshape(-1)
    tbl = jnp.pad(inputs.reshape(_B * _N, _CIN), ((0, 0), (0, _CP - _CIN)))

    mesh = plsc.VectorSubcoreMesh(core_axis_name="c", subcore_axis_name="s",
                                  num_cores=2, num_subcores=16)
    BN = _B * _N
    sc = functools.partial(
        pl.kernel,
        out_type=[
            jax.ShapeDtypeStruct((BN * _S, _CP), jnp.float32),
            jax.ShapeDtypeStruct((BN * _S,), jnp.float32),
            jax.ShapeDtypeStruct((BN * _S,), jnp.float32),
            jax.ShapeDtypeStruct((BN * _S,), jnp.float32),
            jax.ShapeDtypeStruct((BN * _S,), jnp.float32),
        ],
        mesh=mesh,
        compiler_params=pltpu.CompilerParams(needs_layout_passes=False),
        scratch_types=[
            pltpu.VMEM((_N,), jnp.float32),
            pltpu.VMEM((_N,), jnp.float32),
            pltpu.VMEM((_N,), jnp.float32),
            pltpu.VMEM((_N + 16,), jnp.float32),
            pltpu.VMEM((_N + 16,), jnp.int32),
            pltpu.VMEM((_RPW * 16,), jnp.int32),
            pltpu.VMEM((_RPW * 16,), jnp.float32),
            pltpu.VMEM((_RPW * 16,), jnp.float32),
            pltpu.VMEM((_RPW * 16,), jnp.float32),
            pltpu.VMEM((_RPW * 16,), jnp.float32),
            pltpu.VMEM((_GQ * 16, _CP), jnp.float32),
            pltpu.SemaphoreType.DMA,
        ],
    )(_sc_select_gather)
    grouped, relx, rely, relz, seld2 = sc(xf, yf, zf, tbl)

    grouped = grouped.reshape(BN, _S, _CP)
    relx = relx.reshape(BN, _S)
    rely = rely.reshape(BN, _S)
    relz = relz.reshape(BN, _S)
    seld2 = seld2.reshape(BN, _S)

    out = pl.pallas_call(
        _tc_weight_reduce,
        grid=(BN // _NQT,),
        in_specs=[
            pl.BlockSpec((_NQT, _S, _CP), lambda q: (q, 0, 0)),
            pl.BlockSpec((_NQT, _S), lambda q: (q, 0)),
            pl.BlockSpec((_NQT, _S), lambda q: (q, 0)),
            pl.BlockSpec((_NQT, _S), lambda q: (q, 0)),
            pl.BlockSpec((_NQT, _S), lambda q: (q, 0)),
            pl.BlockSpec(memory_space=pltpu.SMEM),
            pl.BlockSpec((_K, _CIN, _F), lambda q: (0, 0, 0)),
        ],
        out_specs=pl.BlockSpec((_NQT, _F), lambda q: (q, 0)),
        out_shape=jax.ShapeDtypeStruct((BN, _F), jnp.float32),
    )(grouped, relx, rely, relz, seld2, kernel_points, W)
    return out.reshape(_B, _N, _F)


# speedup vs baseline: 1.5371x; 1.5371x over previous
import functools

import jax
import jax.numpy as jnp
from jax import lax
from jax.experimental import pallas as pl
from jax.experimental.pallas import tpu as pltpu
from jax.experimental.pallas import tpu_sc as plsc

_B, _N, _CIN = 4, 4096, 67
_S, _K, _F = 16, 15, 64
_KP_EXTEND = 0.1
_ALPHA = 0.3
_R2MAX = (2.0 * _KP_EXTEND) ** 2
_BIG = 1e30
_CP = 128
_NW = 32
_RPW = (_B * _N) // _NW
_WPB = _N // _RPW
_GQ = 8
_NG = _RPW // _GQ
_NCHUNK = _N // 16


def _sc_select_gather(x_hbm, y_hbm, z_hbm, tbl_hbm,
                      grouped_hbm, relx_hbm, rely_hbm, relz_hbm, seld2_hbm,
                      xv, yv, zv, sd2, sidx, idxsel, selk,
                      rxb, ryb, rzb, rows, sem):
    wid = lax.axis_index("s") * 2 + lax.axis_index("c")
    b = wid >> 3
    nbase = (wid & (_WPB - 1)) * _RPW
    gbase = b * _N + nbase

    pltpu.sync_copy(x_hbm.at[pl.ds(b * _N, _N)], xv)
    pltpu.sync_copy(y_hbm.at[pl.ds(b * _N, _N)], yv)
    pltpu.sync_copy(z_hbm.at[pl.ds(b * _N, _N)], zv)

    iota = lax.iota(jnp.int32, 16)

    def row_body(i, carry):
        n = nbase + i
        nsplat = jnp.full((16,), n, jnp.int32)
        qx = plsc.load_gather(xv, [nsplat])
        qy = plsc.load_gather(yv, [nsplat])
        qz = plsc.load_gather(zv, [nsplat])

        def scan_body(c, cnt):
            c16 = c * 16
            xc = xv[pl.ds(c16, 16)]
            yc = yv[pl.ds(c16, 16)]
            zc = zv[pl.ds(c16, 16)]
            dx = xc - qx
            dy = yc - qy
            dz = zc - qz
            d2 = (dx * dx + dy * dy) + dz * dz
            m = d2 < _R2MAX
            pos = plsc.cumsum(m.astype(jnp.int32)) - 1 + cnt
            plsc.store_scatter(sd2, [pos], d2, mask=m)
            plsc.store_scatter(sidx, [pos], iota + c16, mask=m)
            pc = plsc.all_reduce_population_count(m)
            if pc.ndim == 0:
                pc = jnp.full((16,), pc, jnp.int32)
            return cnt + pc

        cnt = lax.fori_loop(0, _NCHUNK, scan_body, jnp.zeros((16,), jnp.int32))
        cnt_s = jnp.max(cnt)

        padpos = cnt + iota
        plsc.store_scatter(sd2, [padpos], jnp.full((16,), _BIG, jnp.float32))
        plsc.store_scatter(sidx, [padpos], nsplat)

        runk, runv = plsc.sort_key_val(sd2[pl.ds(0, 16)], sidx[pl.ds(0, 16)])

        def merge_body(j, kv):
            rk, rv = kv
            bs = plsc.sort_key_val(sd2[pl.ds(j * 16, 16)], sidx[pl.ds(j * 16, 16)])
            rbk = lax.rev(bs[0], (0,))
            rbv = lax.rev(bs[1], (0,))
            keep = rk <= rbk
            mk = jnp.where(keep, rk, rbk)
            mv = jnp.where(keep, rv, rbv)
            sk = plsc.sort_key_val(mk, mv)
            return (sk[0], sk[1])

        nch = (cnt_s + 15) >> 4
        runk, runv = lax.fori_loop(1, nch, merge_body, (runk, runv))

        relx = plsc.load_gather(xv, [runv]) - qx
        rely = plsc.load_gather(yv, [runv]) - qy
        relz = plsc.load_gather(zv, [runv]) - qz

        o = i * 16
        rxb[pl.ds(o, 16)] = relx
        ryb[pl.ds(o, 16)] = rely
        rzb[pl.ds(o, 16)] = relz
        selk[pl.ds(o, 16)] = runk
        idxsel[pl.ds(o, 16)] = runv + b * _N
        return carry

    lax.fori_loop(0, _RPW, row_body, 0)

    ob = gbase * 16
    pltpu.sync_copy(rxb, relx_hbm.at[pl.ds(ob, _RPW * 16)])
    pltpu.sync_copy(ryb, rely_hbm.at[pl.ds(ob, _RPW * 16)])
    pltpu.sync_copy(rzb, relz_hbm.at[pl.ds(ob, _RPW * 16)])
    pltpu.sync_copy(selk, seld2_hbm.at[pl.ds(ob, _RPW * 16)])

    def gather_body(g, carry):
        gi = g * _GQ * 16
        pltpu.async_copy(tbl_hbm.at[idxsel.at[pl.ds(gi, _GQ * 16)]], rows, sem).wait()
        pltpu.sync_copy(rows, grouped_hbm.at[pl.ds(gbase * 16 + gi, _GQ * 16)])
        return carry

    lax.fori_loop(0, _NG, gather_body, 0)


def _tc_weight_reduce(grouped_ref, relx_ref, rely_ref, relz_ref, seld2_ref,
                      kp_ref, w_ref, out_ref):
    g2 = grouped_ref[...]
    relx = relx_ref[...]
    rely = rely_ref[...]
    relz = relz_ref[...]
    validf = jnp.where(seld2_ref[...] < _R2MAX, 1.0, 0.0)

    nq = relx.shape[0]
    out = jnp.zeros((nq, _F), jnp.float32)
    for k in range(_K):
        ax, ay, az = kp_ref[k, 0], kp_ref[k, 1], kp_ref[k, 2]
        dxk = relx - ax
        dyk = rely - ay
        dzk = relz - az
        dk2 = (dxk * dxk + dyk * dyk) + dzk * dzk
        dist = jnp.sqrt(dk2 + 1e-12)
        infl = jnp.maximum(0.0, 1.0 - dist * (1.0 / _KP_EXTEND)) * validf
        w3 = g2.reshape(nq, _S, _CP) * infl[:, :, None]
        r8 = w3[:, 0:8, :] + w3[:, 8:16, :]
        r4 = r8[:, 0:4, :] + r8[:, 4:8, :]
        r2 = r4[:, 0:2, :] + r4[:, 2:4, :]
        hk = r2[:, 0, :] + r2[:, 1, :]
        out = out + jax.lax.dot_general(hk[:, :_CIN], w_ref[k], (((1,), (0,)), ((), ())),
                                        preferred_element_type=jnp.float32)
    out_ref[...] = jnp.where(out > 0, out, _ALPHA * out)


_NQT = 128


def kernel(inputs, kernel_points, W):
    xyz = inputs[..., :3]
    xf = xyz[..., 0].reshape(-1)
    yf = xyz[..., 1].reshape(-1)
    zf = xyz[..., 2].reshape(-1)
    tbl = jnp.pad(inputs.reshape(_B * _N, _CIN), ((0, 0), (0, _CP - _CIN)))

    mesh = plsc.VectorSubcoreMesh(core_axis_name="c", subcore_axis_name="s",
                                  num_cores=2, num_subcores=16)
    BN = _B * _N
    sc = functools.partial(
        pl.kernel,
        out_type=[
            jax.ShapeDtypeStruct((BN * _S, _CP), jnp.float32),
            jax.ShapeDtypeStruct((BN * _S,), jnp.float32),
            jax.ShapeDtypeStruct((BN * _S,), jnp.float32),
            jax.ShapeDtypeStruct((BN * _S,), jnp.float32),
            jax.ShapeDtypeStruct((BN * _S,), jnp.float32),
        ],
        mesh=mesh,
        compiler_params=pltpu.CompilerParams(needs_layout_passes=False),
        scratch_types=[
            pltpu.VMEM((_N,), jnp.float32),
            pltpu.VMEM((_N,), jnp.float32),
            pltpu.VMEM((_N,), jnp.float32),
            pltpu.VMEM((_N + 16,), jnp.float32),
            pltpu.VMEM((_N + 16,), jnp.int32),
            pltpu.VMEM((_RPW * 16,), jnp.int32),
            pltpu.VMEM((_RPW * 16,), jnp.float32),
            pltpu.VMEM((_RPW * 16,), jnp.float32),
            pltpu.VMEM((_RPW * 16,), jnp.float32),
            pltpu.VMEM((_RPW * 16,), jnp.float32),
            pltpu.VMEM((_GQ * 16, _CP), jnp.float32),
            pltpu.SemaphoreType.DMA,
        ],
    )(_sc_select_gather)
    grouped, relx, rely, relz, seld2 = sc(xf, yf, zf, tbl)

    relx = relx.reshape(BN, _S)
    rely = rely.reshape(BN, _S)
    relz = relz.reshape(BN, _S)
    seld2 = seld2.reshape(BN, _S)

    out = pl.pallas_call(
        _tc_weight_reduce,
        grid=(BN // _NQT,),
        in_specs=[
            pl.BlockSpec((_NQT * _S, _CP), lambda q: (q, 0)),
            pl.BlockSpec((_NQT, _S), lambda q: (q, 0)),
            pl.BlockSpec((_NQT, _S), lambda q: (q, 0)),
            pl.BlockSpec((_NQT, _S), lambda q: (q, 0)),
            pl.BlockSpec((_NQT, _S), lambda q: (q, 0)),
            pl.BlockSpec(memory_space=pltpu.SMEM),
            pl.BlockSpec((_K, _CIN, _F), lambda q: (0, 0, 0)),
        ],
        out_specs=pl.BlockSpec((_NQT, _F), lambda q: (q, 0)),
        out_shape=jax.ShapeDtypeStruct((BN, _F), jnp.float32),
    )(grouped, relx, rely, relz, seld2, kernel_points, W)
    return out.reshape(_B, _N, _F)


# speedup vs baseline: 3.5854x; 2.3326x over previous
import functools

import jax
import jax.numpy as jnp
from jax import lax
from jax.experimental import pallas as pl
from jax.experimental.pallas import tpu as pltpu
from jax.experimental.pallas import tpu_sc as plsc

_B, _N, _CIN = 4, 4096, 67
_S, _K, _F = 16, 15, 64
_KP_EXTEND = 0.1
_ALPHA = 0.3
_R2MAX = (2.0 * _KP_EXTEND) ** 2
_BIG = 1e30
_CP = 128
_NW = 32
_RPW = (_B * _N) // _NW
_WPB = _N // _RPW
_GQ = 8
_NG = _RPW // _GQ
_NCHUNK = _N // 16


def _sc_select_gather(x_hbm, y_hbm, z_hbm, tbl_hbm,
                      grouped_hbm, relx_hbm, rely_hbm, relz_hbm, seld2_hbm,
                      xv, yv, zv, sd2, sidx, idxsel, selk,
                      rxb, ryb, rzb, rows, sem):
    wid = lax.axis_index("s") * 2 + lax.axis_index("c")
    b = wid >> 3
    nbase = (wid & (_WPB - 1)) * _RPW
    gbase = b * _N + nbase

    pltpu.sync_copy(x_hbm.at[pl.ds(b * _N, _N)], xv)
    pltpu.sync_copy(y_hbm.at[pl.ds(b * _N, _N)], yv)
    pltpu.sync_copy(z_hbm.at[pl.ds(b * _N, _N)], zv)

    iota = lax.iota(jnp.int32, 16)

    def row_body(i, carry):
        n = nbase + i
        nsplat = jnp.full((16,), n, jnp.int32)
        qx = plsc.load_gather(xv, [nsplat])
        qy = plsc.load_gather(yv, [nsplat])
        qz = plsc.load_gather(zv, [nsplat])

        def scan_body(c16, cnt):
            xc = xv[pl.ds(c16, 16)]
            yc = yv[pl.ds(c16, 16)]
            zc = zv[pl.ds(c16, 16)]
            dx = xc - qx
            dy = yc - qy
            dz = zc - qz
            d2 = (dx * dx + dy * dy) + dz * dz
            m = d2 < _R2MAX
            pos = plsc.cumsum(m.astype(jnp.int32)) - 1 + cnt
            plsc.store_scatter(sd2, [pos], d2, mask=m)
            plsc.store_scatter(sidx, [pos], iota + c16, mask=m)
            pc = plsc.all_reduce_population_count(m)
            if pc.ndim == 0:
                pc = jnp.full((16,), pc, jnp.int32)
            return cnt + pc

        cnt = plsc.parallel_loop(0, _N, step=16,
                                 carry=jnp.zeros((16,), jnp.int32))(scan_body)
        cnt_s = jnp.max(cnt)

        padpos = cnt + iota
        plsc.store_scatter(sd2, [padpos], jnp.full((16,), _BIG, jnp.float32))
        plsc.store_scatter(sidx, [padpos], nsplat)

        runk, runv = plsc.sort_key_val(sd2[pl.ds(0, 16)], sidx[pl.ds(0, 16)])

        def merge_body(j, kv):
            rk, rv = kv
            bs = plsc.sort_key_val(sd2[pl.ds(j * 16, 16)], sidx[pl.ds(j * 16, 16)])
            rbk = lax.rev(bs[0], (0,))
            rbv = lax.rev(bs[1], (0,))
            keep = rk <= rbk
            mk = jnp.where(keep, rk, rbk)
            mv = jnp.where(keep, rv, rbv)
            sk = plsc.sort_key_val(mk, mv)
            return (sk[0], sk[1])

        nch = (cnt_s + 15) >> 4
        runk, runv = lax.fori_loop(1, nch, merge_body, (runk, runv))

        relx = plsc.load_gather(xv, [runv]) - qx
        rely = plsc.load_gather(yv, [runv]) - qy
        relz = plsc.load_gather(zv, [runv]) - qz

        o = i * 16
        rxb[pl.ds(o, 16)] = relx
        ryb[pl.ds(o, 16)] = rely
        rzb[pl.ds(o, 16)] = relz
        selk[pl.ds(o, 16)] = runk
        idxsel[pl.ds(o, 16)] = runv + b * _N
        return carry

    lax.fori_loop(0, _RPW, row_body, 0)

    ob = gbase * 16
    pltpu.sync_copy(rxb, relx_hbm.at[pl.ds(ob, _RPW * 16)])
    pltpu.sync_copy(ryb, rely_hbm.at[pl.ds(ob, _RPW * 16)])
    pltpu.sync_copy(rzb, relz_hbm.at[pl.ds(ob, _RPW * 16)])
    pltpu.sync_copy(selk, seld2_hbm.at[pl.ds(ob, _RPW * 16)])

    def gather_body(g, carry):
        gi = g * _GQ * 16
        pltpu.async_copy(tbl_hbm.at[idxsel.at[pl.ds(gi, _GQ * 16)]], rows, sem).wait()
        pltpu.sync_copy(rows, grouped_hbm.at[pl.ds(gbase * 16 + gi, _GQ * 16)])
        return carry

    lax.fori_loop(0, _NG, gather_body, 0)


def _tc_weight_reduce(grouped_ref, relx_ref, rely_ref, relz_ref, seld2_ref,
                      kp_ref, w_ref, out_ref):
    g2 = grouped_ref[...]
    relx = relx_ref[...]
    rely = rely_ref[...]
    relz = relz_ref[...]
    validf = jnp.where(seld2_ref[...] < _R2MAX, 1.0, 0.0)

    nq = relx.shape[0]
    out = jnp.zeros((nq, _F), jnp.float32)
    for k in range(_K):
        ax, ay, az = kp_ref[k, 0], kp_ref[k, 1], kp_ref[k, 2]
        dxk = relx - ax
        dyk = rely - ay
        dzk = relz - az
        dk2 = (dxk * dxk + dyk * dyk) + dzk * dzk
        dist = jnp.sqrt(dk2 + 1e-12)
        infl = jnp.maximum(0.0, 1.0 - dist * (1.0 / _KP_EXTEND)) * validf
        w3 = g2.reshape(nq, _S, _CP) * infl[:, :, None]
        r8 = w3[:, 0:8, :] + w3[:, 8:16, :]
        r4 = r8[:, 0:4, :] + r8[:, 4:8, :]
        r2 = r4[:, 0:2, :] + r4[:, 2:4, :]
        hk = r2[:, 0, :] + r2[:, 1, :]
        out = out + jax.lax.dot_general(hk[:, :_CIN], w_ref[k], (((1,), (0,)), ((), ())),
                                        preferred_element_type=jnp.float32)
    out_ref[...] = jnp.where(out > 0, out, _ALPHA * out)


_NQT = 128


def kernel(inputs, kernel_points, W):
    xyz = inputs[..., :3]
    xf = xyz[..., 0].reshape(-1)
    yf = xyz[..., 1].reshape(-1)
    zf = xyz[..., 2].reshape(-1)
    tbl = jnp.pad(inputs.reshape(_B * _N, _CIN), ((0, 0), (0, _CP - _CIN)))

    mesh = plsc.VectorSubcoreMesh(core_axis_name="c", subcore_axis_name="s",
                                  num_cores=2, num_subcores=16)
    BN = _B * _N
    sc = functools.partial(
        pl.kernel,
        out_type=[
            jax.ShapeDtypeStruct((BN * _S, _CP), jnp.float32),
            jax.ShapeDtypeStruct((BN * _S,), jnp.float32),
            jax.ShapeDtypeStruct((BN * _S,), jnp.float32),
            jax.ShapeDtypeStruct((BN * _S,), jnp.float32),
            jax.ShapeDtypeStruct((BN * _S,), jnp.float32),
        ],
        mesh=mesh,
        compiler_params=pltpu.CompilerParams(needs_layout_passes=False),
        scratch_types=[
            pltpu.VMEM((_N,), jnp.float32),
            pltpu.VMEM((_N,), jnp.float32),
            pltpu.VMEM((_N,), jnp.float32),
            pltpu.VMEM((_N + 16,), jnp.float32),
            pltpu.VMEM((_N + 16,), jnp.int32),
            pltpu.VMEM((_RPW * 16,), jnp.int32),
            pltpu.VMEM((_RPW * 16,), jnp.float32),
            pltpu.VMEM((_RPW * 16,), jnp.float32),
            pltpu.VMEM((_RPW * 16,), jnp.float32),
            pltpu.VMEM((_RPW * 16,), jnp.float32),
            pltpu.VMEM((_GQ * 16, _CP), jnp.float32),
            pltpu.SemaphoreType.DMA,
        ],
    )(_sc_select_gather)
    grouped, relx, rely, relz, seld2 = sc(xf, yf, zf, tbl)

    relx = relx.reshape(BN, _S)
    rely = rely.reshape(BN, _S)
    relz = relz.reshape(BN, _S)
    seld2 = seld2.reshape(BN, _S)

    out = pl.pallas_call(
        _tc_weight_reduce,
        grid=(BN // _NQT,),
        in_specs=[
            pl.BlockSpec((_NQT * _S, _CP), lambda q: (q, 0)),
            pl.BlockSpec((_NQT, _S), lambda q: (q, 0)),
            pl.BlockSpec((_NQT, _S), lambda q: (q, 0)),
            pl.BlockSpec((_NQT, _S), lambda q: (q, 0)),
            pl.BlockSpec((_NQT, _S), lambda q: (q, 0)),
            pl.BlockSpec(memory_space=pltpu.SMEM),
            pl.BlockSpec((_K, _CIN, _F), lambda q: (0, 0, 0)),
        ],
        out_specs=pl.BlockSpec((_NQT, _F), lambda q: (q, 0)),
        out_shape=jax.ShapeDtypeStruct((BN, _F), jnp.float32),
    )(grouped, relx, rely, relz, seld2, kernel_points, W)
    return out.reshape(_B, _N, _F)
